# baseline (speedup 1.0000x reference)
import jax
import jax.numpy as jnp
from jax.experimental import pallas as pl
from jax.experimental.pallas import tpu as pltpu

_B, _D, _H, _W = 16, 512, 32, 32
_N = _H * _W
_M = 10
_NB = 4
_NT = _N // _NB


def _body(q_ref, k_ref, uq_ref, ls_ref, lc_ref, ci_ref, um_ref,
          kk_ref, qu_ref, cm_ref):
    b = pl.program_id(0)
    nb = pl.program_id(1)
    x = q_ref[0]
    keys = k_ref[...]

    ss = jnp.sum(x * x, axis=0, keepdims=True)
    qn = x / jnp.maximum(jnp.sqrt(ss), 1e-12)

    score = jax.lax.dot_general(
        keys, qn, (((1,), (0,)), ((), ())),
        preferred_element_type=jnp.float32)

    rmax = jnp.max(score, axis=0, keepdims=True)
    e = jnp.exp(score - rmax)
    score_memory = e / jnp.sum(e, axis=0, keepdims=True)

    row_ids = jax.lax.broadcasted_iota(jnp.int32, (_M, _NT), 0)
    gidx = jnp.min(jnp.where(score == rmax, row_ids, _M), axis=0,
                   keepdims=True)
    oh1 = (row_ids == gidx)
    score2 = jnp.where(oh1, -jnp.inf, score)
    rmax2 = jnp.max(score2, axis=0, keepdims=True)
    gidx2 = jnp.min(jnp.where(score2 == rmax2, row_ids, _M), axis=0,
                    keepdims=True)
    oh1f = oh1.astype(jnp.float32)
    oh2f = (row_ids == gidx2).astype(jnp.float32)

    pos = jax.lax.dot_general(keys, oh1f, (((0,), (0,)), ((), ())),
                              preferred_element_type=jnp.float32)
    neg = jax.lax.dot_general(keys, oh2f, (((0,), (0,)), ((), ())),
                              preferred_element_type=jnp.float32)
    cat = jax.lax.dot_general(keys, score_memory, (((0,), (0,)), ((), ())),
                              preferred_element_type=jnp.float32)

    diff = qn - pos
    lc_ref[0] = jnp.transpose(diff * diff)

    dpe = diff + 1e-6
    dne = (qn - neg) + 1e-6
    dp = jnp.sqrt(jnp.sum(dpe * dpe, axis=0, keepdims=True))
    dn = jnp.sqrt(jnp.sum(dne * dne, axis=0, keepdims=True))
    ls_ref[0] = jnp.maximum(dp - dn + 1.0, 0.0)
    ci_ref[0] = gidx

    uq_ref[0, 0:_D, :] = qn
    uq_ref[0, _D:2 * _D, :] = cat

    bmax = jnp.max(score, axis=1, keepdims=True)

    @pl.when(nb == 0)
    def _first():
        masked = jnp.exp(score - bmax) * oh1f
        qu_ref[...] = jax.lax.dot_general(
            masked, qn, (((1,), (1,)), ((), ())),
            preferred_element_type=jnp.float32)
        cm_ref[...] = bmax

    @pl.when(nb != 0)
    def _rest():
        cm_old = cm_ref[...]
        cm_new = jnp.maximum(cm_old, bmax)
        masked = jnp.exp(score - cm_new) * oh1f
        part = jax.lax.dot_general(
            masked, qn, (((1,), (1,)), ((), ())),
            preferred_element_type=jnp.float32)
        qu_ref[...] = qu_ref[...] * jnp.exp(cm_old - cm_new) + part
        cm_ref[...] = cm_new

    @pl.when(nb == _NB - 1)
    def _update():
        @pl.when(b == 0)
        def _init():
            kk_ref[...] = keys

        s = qu_ref[...] + kk_ref[...]
        nrm = jnp.sqrt(jnp.sum(s * s, axis=1, keepdims=True))
        kk = s / jnp.maximum(nrm, 1e-12)
        kk_ref[...] = kk

        @pl.when(b == _B - 1)
        def _fin():
            um_ref[...] = kk


def kernel(query, keys):
    qv = query.reshape(_B, _D, _N)
    uq, ls, lc, ci, um = pl.pallas_call(
        _body,
        grid=(_B, _NB),
        in_specs=[
            pl.BlockSpec((1, _D, _NT), lambda b, nb: (b, 0, nb)),
            pl.BlockSpec((_M, _D), lambda b, nb: (0, 0)),
        ],
        out_specs=[
            pl.BlockSpec((1, 2 * _D, _NT), lambda b, nb: (b, 0, nb)),
            pl.BlockSpec((1, 1, _NT), lambda b, nb: (b, 0, nb)),
            pl.BlockSpec((1, _NT, _D), lambda b, nb: (b, nb, 0)),
            pl.BlockSpec((1, 1, _NT), lambda b, nb: (b, 0, nb)),
            pl.BlockSpec((_M, _D), lambda b, nb: (0, 0)),
        ],
        out_shape=[
            jax.ShapeDtypeStruct((_B, 2 * _D, _N), jnp.float32),
            jax.ShapeDtypeStruct((_B, 1, _N), jnp.float32),
            jax.ShapeDtypeStruct((_B, _N, _D), jnp.float32),
            jax.ShapeDtypeStruct((_B, 1, _N), jnp.int32),
            jax.ShapeDtypeStruct((_M, _D), jnp.float32),
        ],
        scratch_shapes=[
            pltpu.VMEM((_M, _D), jnp.float32),
            pltpu.VMEM((_M, _D), jnp.float32),
            pltpu.VMEM((_M, 1), jnp.float32),
        ],
        compiler_params=pltpu.CompilerParams(
            dimension_semantics=("arbitrary", "arbitrary")),
    )(qv, keys)
    updated_query = uq.reshape(_B, 2 * _D, _H, _W)
    return (updated_query, um, ls.reshape(_B, _N), lc,
            ci.reshape(_B, _N))


# speedup vs baseline: 1.2779x; 1.2779x over previous
import jax
import jax.numpy as jnp
from jax.experimental import pallas as pl
from jax.experimental.pallas import tpu as pltpu

_B, _D, _H, _W = 16, 512, 32, 32
_N = _H * _W
_M = 10
_BB = 2
_G = _B // _BB


def _body(q_ref, k_ref, uq_ref, ls_ref, lc_ref, ci_ref, um_ref, kk_ref):
    g = pl.program_id(0)
    keys = k_ref[...]

    @pl.when(g == 0)
    def _init():
        kk_ref[...] = keys

    for s in range(_BB):
        x = q_ref[s]

        ss = jnp.sum(x * x, axis=0, keepdims=True)
        qn = x / jnp.maximum(jnp.sqrt(ss), 1e-12)

        score = jax.lax.dot_general(
            keys, qn, (((1,), (0,)), ((), ())),
            preferred_element_type=jnp.float32)

        rmax = jnp.max(score, axis=0, keepdims=True)
        e = jnp.exp(score - rmax)
        score_memory = e / jnp.sum(e, axis=0, keepdims=True)

        row_ids = jax.lax.broadcasted_iota(jnp.int32, (_M, _N), 0)
        gidx = jnp.min(jnp.where(score == rmax, row_ids, _M), axis=0,
                       keepdims=True)
        oh1 = (row_ids == gidx)
        score2 = jnp.where(oh1, -jnp.inf, score)
        rmax2 = jnp.max(score2, axis=0, keepdims=True)
        gidx2 = jnp.min(jnp.where(score2 == rmax2, row_ids, _M), axis=0,
                        keepdims=True)
        oh1f = oh1.astype(jnp.float32)
        oh2f = (row_ids == gidx2).astype(jnp.float32)

        pos = jax.lax.dot_general(keys, oh1f, (((0,), (0,)), ((), ())),
                                  preferred_element_type=jnp.float32)
        neg = jax.lax.dot_general(keys, oh2f, (((0,), (0,)), ((), ())),
                                  preferred_element_type=jnp.float32)
        cat = jax.lax.dot_general(keys, score_memory,
                                  (((0,), (0,)), ((), ())),
                                  preferred_element_type=jnp.float32)

        diff = qn - pos
        lc_ref[s] = jnp.transpose(diff * diff)

        dpe = diff + 1e-6
        dne = (qn - neg) + 1e-6
        dp = jnp.sqrt(jnp.sum(dpe * dpe, axis=0, keepdims=True))
        dn = jnp.sqrt(jnp.sum(dne * dne, axis=0, keepdims=True))
        ls_ref[s] = jnp.maximum(dp - dn + 1.0, 0.0)
        ci_ref[s] = gidx

        uq_ref[s, 0:_D, :] = qn
        uq_ref[s, _D:2 * _D, :] = cat

        cmax = jnp.max(score, axis=1, keepdims=True)
        masked = jnp.exp(score - cmax) * oh1f
        qu = jax.lax.dot_general(masked, qn, (((1,), (1,)), ((), ())),
                                 preferred_element_type=jnp.float32)

        t = qu + kk_ref[...]
        nrm = jnp.sqrt(jnp.sum(t * t, axis=1, keepdims=True))
        kk_ref[...] = t / jnp.maximum(nrm, 1e-12)

    @pl.when(g == _G - 1)
    def _fin():
        um_ref[...] = kk_ref[...]


def kernel(query, keys):
    qv = query.reshape(_B, _D, _N)
    uq, ls, lc, ci, um = pl.pallas_call(
        _body,
        grid=(_G,),
        in_specs=[
            pl.BlockSpec((_BB, _D, _N), lambda g: (g, 0, 0)),
            pl.BlockSpec((_M, _D), lambda g: (0, 0)),
        ],
        out_specs=[
            pl.BlockSpec((_BB, 2 * _D, _N), lambda g: (g, 0, 0)),
            pl.BlockSpec((_BB, 1, _N), lambda g: (g, 0, 0)),
            pl.BlockSpec((_BB, _N, _D), lambda g: (g, 0, 0)),
            pl.BlockSpec((_BB, 1, _N), lambda g: (g, 0, 0)),
            pl.BlockSpec((_M, _D), lambda g: (0, 0)),
        ],
        out_shape=[
            jax.ShapeDtypeStruct((_B, 2 * _D, _N), jnp.float32),
            jax.ShapeDtypeStruct((_B, 1, _N), jnp.float32),
            jax.ShapeDtypeStruct((_B, _N, _D), jnp.float32),
            jax.ShapeDtypeStruct((_B, 1, _N), jnp.int32),
            jax.ShapeDtypeStruct((_M, _D), jnp.float32),
        ],
        scratch_shapes=[pltpu.VMEM((_M, _D), jnp.float32)],
        compiler_params=pltpu.CompilerParams(
            dimension_semantics=("arbitrary",)),
    )(qv, keys)
    updated_query = uq.reshape(_B, 2 * _D, _H, _W)
    return (updated_query, um, ls.reshape(_B, _N), lc,
            ci.reshape(_B, _N))


# speedup vs baseline: 1.2925x; 1.0114x over previous
import jax
import jax.numpy as jnp
from jax.experimental import pallas as pl
from jax.experimental.pallas import tpu as pltpu

_B, _D, _H, _W = 16, 512, 32, 32
_N = _H * _W
_M = 10
_BB = 2
_G = _B // _BB


def _body(q_ref, k_ref, uq_ref, ls_ref, lc_ref, ci_ref, um_ref, kk_ref):
    g = pl.program_id(0)
    keys = k_ref[...]

    @pl.when(g == 0)
    def _init():
        kk_ref[...] = keys

    for s in range(_BB):
        x = q_ref[s]

        ss = jnp.sum(x * x, axis=0, keepdims=True)
        qn = x / jnp.maximum(jnp.sqrt(ss), 1e-12)

        score = jax.lax.dot_general(
            keys, qn, (((1,), (0,)), ((), ())),
            preferred_element_type=jnp.float32)

        rmax = jnp.max(score, axis=0, keepdims=True)
        e = jnp.exp(score - rmax)
        score_memory = e / jnp.sum(e, axis=0, keepdims=True)

        row_ids = jax.lax.broadcasted_iota(jnp.int32, (_M, _N), 0)
        gidx = jnp.min(jnp.where(score == rmax, row_ids, _M), axis=0,
                       keepdims=True)
        oh1 = (row_ids == gidx)
        score2 = jnp.where(oh1, -jnp.inf, score)
        rmax2 = jnp.max(score2, axis=0, keepdims=True)
        gidx2 = jnp.min(jnp.where(score2 == rmax2, row_ids, _M), axis=0,
                        keepdims=True)
        oh1f = oh1.astype(jnp.float32)
        oh2f = (row_ids == gidx2).astype(jnp.float32)

        pos = jax.lax.dot_general(keys, oh1f, (((0,), (0,)), ((), ())),
                                  preferred_element_type=jnp.float32)
        cat = jax.lax.dot_general(keys, score_memory,
                                  (((0,), (0,)), ((), ())),
                                  preferred_element_type=jnp.float32)

        diff = qn - pos
        lc_ref[s] = jnp.transpose(diff * diff)

        eps = 1e-6
        k2 = jnp.sum(keys * keys, axis=1, keepdims=True)
        ks = jnp.sum(keys, axis=1, keepdims=True)
        qs = jnp.sum(qn, axis=0, keepdims=True)
        k2g1 = jnp.sum(oh1f * k2, axis=0, keepdims=True)
        ksg1 = jnp.sum(oh1f * ks, axis=0, keepdims=True)
        k2g2 = jnp.sum(oh2f * k2, axis=0, keepdims=True)
        ksg2 = jnp.sum(oh2f * ks, axis=0, keepdims=True)
        de2 = _D * eps * eps
        dp2 = 1.0 - 2.0 * rmax + k2g1 + 2.0 * eps * (qs - ksg1) + de2
        dn2 = 1.0 - 2.0 * rmax2 + k2g2 + 2.0 * eps * (qs - ksg2) + de2
        dp = jnp.sqrt(jnp.maximum(dp2, 0.0))
        dn = jnp.sqrt(jnp.maximum(dn2, 0.0))
        ls_ref[s] = jnp.maximum(dp - dn + 1.0, 0.0)
        ci_ref[s] = gidx

        uq_ref[s, 0:_D, :] = qn
        uq_ref[s, _D:2 * _D, :] = cat

        cmax = jnp.max(score, axis=1, keepdims=True)
        masked = jnp.exp(score - cmax) * oh1f
        qu = jax.lax.dot_general(masked, qn, (((1,), (1,)), ((), ())),
                                 preferred_element_type=jnp.float32)

        t = qu + kk_ref[...]
        nrm = jnp.sqrt(jnp.sum(t * t, axis=1, keepdims=True))
        kk_ref[...] = t / jnp.maximum(nrm, 1e-12)

    @pl.when(g == _G - 1)
    def _fin():
        um_ref[...] = kk_ref[...]


def kernel(query, keys):
    qv = query.reshape(_B, _D, _N)
    uq, ls, lc, ci, um = pl.pallas_call(
        _body,
        grid=(_G,),
        in_specs=[
            pl.BlockSpec((_BB, _D, _N), lambda g: (g, 0, 0)),
            pl.BlockSpec((_M, _D), lambda g: (0, 0)),
        ],
        out_specs=[
            pl.BlockSpec((_BB, 2 * _D, _N), lambda g: (g, 0, 0)),
            pl.BlockSpec((_BB, 1, _N), lambda g: (g, 0, 0)),
            pl.BlockSpec((_BB, _N, _D), lambda g: (g, 0, 0)),
            pl.BlockSpec((_BB, 1, _N), lambda g: (g, 0, 0)),
            pl.BlockSpec((_M, _D), lambda g: (0, 0)),
        ],
        out_shape=[
            jax.ShapeDtypeStruct((_B, 2 * _D, _N), jnp.float32),
            jax.ShapeDtypeStruct((_B, 1, _N), jnp.float32),
            jax.ShapeDtypeStruct((_B, _N, _D), jnp.float32),
            jax.ShapeDtypeStruct((_B, 1, _N), jnp.int32),
            jax.ShapeDtypeStruct((_M, _D), jnp.float32),
        ],
        scratch_shapes=[pltpu.VMEM((_M, _D), jnp.float32)],
        compiler_params=pltpu.CompilerParams(
            dimension_semantics=("arbitrary",)),
    )(qv, keys)
    updated_query = uq.reshape(_B, 2 * _D, _H, _W)
    return (updated_query, um, ls.reshape(_B, _N), lc,
            ci.reshape(_B, _N))


# speedup vs baseline: 1.2975x; 1.0038x over previous
import jax
import jax.numpy as jnp
from jax.experimental import pallas as pl
from jax.experimental.pallas import tpu as pltpu

_B, _D, _H, _W = 16, 512, 32, 32
_N = _H * _W
_M = 10
_BB = 2
_G = _B // _BB


def _body(q_ref, k_ref, uq_ref, ls_ref, lc_ref, ci_ref, um_ref, kk_ref):
    g = pl.program_id(0)
    keys = k_ref[...]

    @pl.when(g == 0)
    def _init():
        kk_ref[...] = keys

    for s in range(_BB):
        x = q_ref[s]
        ones = jnp.ones((1, _D), jnp.float32)

        ss = jax.lax.dot_general(ones, x * x, (((1,), (0,)), ((), ())),
                                 preferred_element_type=jnp.float32)
        rsn = 1.0 / jnp.maximum(jnp.sqrt(ss), 1e-12)
        qn = x * rsn

        ka = jnp.concatenate([keys, ones], axis=0)
        sq = jax.lax.dot_general(
            ka, x, (((1,), (0,)), ((), ())),
            preferred_element_type=jnp.float32) * rsn
        score = sq[0:_M]
        qs = sq[_M:_M + 1]

        rmax = jnp.max(score, axis=0, keepdims=True)
        e = jnp.exp(score - rmax)
        score_memory = e / jnp.sum(e, axis=0, keepdims=True)

        row_ids = jax.lax.broadcasted_iota(jnp.int32, (_M, _N), 0)
        gidx = jnp.min(jnp.where(score == rmax, row_ids, _M), axis=0,
                       keepdims=True)
        oh1 = (row_ids == gidx)
        score2 = jnp.where(oh1, -jnp.inf, score)
        rmax2 = jnp.max(score2, axis=0, keepdims=True)
        gidx2 = jnp.min(jnp.where(score2 == rmax2, row_ids, _M), axis=0,
                        keepdims=True)
        oh1f = oh1.astype(jnp.float32)
        oh2f = (row_ids == gidx2).astype(jnp.float32)

        rhs = jnp.concatenate([oh1f, score_memory], axis=1)
        pc = jax.lax.dot_general(keys, rhs, (((0,), (0,)), ((), ())),
                                 preferred_element_type=jnp.float32)
        pos = pc[:, 0:_N]
        cat = pc[:, _N:2 * _N]

        diff = qn - pos
        lc_ref[s] = jnp.transpose(diff * diff)

        eps = 1e-6
        k2 = jnp.sum(keys * keys, axis=1, keepdims=True)
        ks = jnp.sum(keys, axis=1, keepdims=True)
        k2g1 = jnp.sum(oh1f * k2, axis=0, keepdims=True)
        ksg1 = jnp.sum(oh1f * ks, axis=0, keepdims=True)
        k2g2 = jnp.sum(oh2f * k2, axis=0, keepdims=True)
        ksg2 = jnp.sum(oh2f * ks, axis=0, keepdims=True)
        de2 = _D * eps * eps
        dp2 = 1.0 - 2.0 * rmax + k2g1 + 2.0 * eps * (qs - ksg1) + de2
        dn2 = 1.0 - 2.0 * rmax2 + k2g2 + 2.0 * eps * (qs - ksg2) + de2
        dp = jnp.sqrt(jnp.maximum(dp2, 0.0))
        dn = jnp.sqrt(jnp.maximum(dn2, 0.0))
        ls_ref[s] = jnp.maximum(dp - dn + 1.0, 0.0)
        ci_ref[s] = gidx

        uq_ref[s, 0:_D, :] = qn
        uq_ref[s, _D:2 * _D, :] = cat

        cmax = jnp.max(score, axis=1, keepdims=True)
        masked = jnp.exp(score - cmax) * oh1f
        qu = jax.lax.dot_general(masked, qn, (((1,), (1,)), ((), ())),
                                 preferred_element_type=jnp.float32)

        t = qu + kk_ref[...]
        nrm = jnp.sqrt(jnp.sum(t * t, axis=1, keepdims=True))
        kk_ref[...] = t / jnp.maximum(nrm, 1e-12)

    @pl.when(g == _G - 1)
    def _fin():
        um_ref[...] = kk_ref[...]


def kernel(query, keys):
    qv = query.reshape(_B, _D, _N)
    uq, ls, lc, ci, um = pl.pallas_call(
        _body,
        grid=(_G,),
        in_specs=[
            pl.BlockSpec((_BB, _D, _N), lambda g: (g, 0, 0)),
            pl.BlockSpec((_M, _D), lambda g: (0, 0)),
        ],
        out_specs=[
            pl.BlockSpec((_BB, 2 * _D, _N), lambda g: (g, 0, 0)),
            pl.BlockSpec((_BB, 1, _N), lambda g: (g, 0, 0)),
            pl.BlockSpec((_BB, _N, _D), lambda g: (g, 0, 0)),
            pl.BlockSpec((_BB, 1, _N), lambda g: (g, 0, 0)),
            pl.BlockSpec((_M, _D), lambda g: (0, 0)),
        ],
        out_shape=[
            jax.ShapeDtypeStruct((_B, 2 * _D, _N), jnp.float32),
            jax.ShapeDtypeStruct((_B, 1, _N), jnp.float32),
            jax.ShapeDtypeStruct((_B, _N, _D), jnp.float32),
            jax.ShapeDtypeStruct((_B, 1, _N), jnp.int32),
            jax.ShapeDtypeStruct((_M, _D), jnp.float32),
        ],
        scratch_shapes=[pltpu.VMEM((_M, _D), jnp.float32)],
        compiler_params=pltpu.CompilerParams(
            dimension_semantics=("arbitrary",)),
    )(qv, keys)
    updated_query = uq.reshape(_B, 2 * _D, _H, _W)
    return (updated_query, um, ls.reshape(_B, _N), lc,
            ci.reshape(_B, _N))
